# baseline (speedup 1.0000x reference)
import jax
import jax.numpy as jnp
from jax import lax
from jax.experimental import pallas as pl
from jax.experimental.pallas import tpu as pltpu
from jax.experimental.pallas import tpu_sc as plsc

N = 10000
E = 320000
D = 128
EPS = 1e-5

NC = 2
NS = 16
NW = NC * NS
EDGES_PER_W = E // NW
K = 112
PAD = 80
NCHUNK = (EDGES_PER_W + PAD) // K
NBUF = 3
RSUB = 624


def _agg_body(x_hbm, xseed_hbm, rc_hbm, zero_hbm, out_hbm,
              rcbuf, gbuf, rcsem, gsem, agg_sp):
    c = lax.axis_index("c")
    s = lax.axis_index("s")
    wid = c * NS + s

    @pl.when((c == 0) & (s == 0))
    def _():
        pltpu.sync_copy(xseed_hbm, agg_sp)

    @pl.when((c == 1) & (s == 0))
    def _():
        pltpu.sync_copy(zero_hbm, agg_sp)

    plsc.subcore_barrier()

    pltpu.sync_copy(rc_hbm.at[wid, 0], rcbuf.at[0])
    pltpu.async_copy(x_hbm.at[rcbuf.at[0, 1]], gbuf.at[0], gsem.at[0])
    pltpu.async_copy(rc_hbm.at[wid, 1], rcbuf.at[1], rcsem.at[1])

    def body(j, carry):
        b0 = lax.rem(j, NBUF)
        b1 = lax.rem(j + 1, NBUF)
        b2 = lax.rem(j + 2, NBUF)

        @pl.when(j + 2 < NCHUNK)
        def _():
            pltpu.async_copy(rc_hbm.at[wid, j + 2], rcbuf.at[b2],
                             rcsem.at[b2])

        @pl.when(j + 1 < NCHUNK)
        def _():
            pltpu.make_async_copy(rc_hbm.at[wid, j + 1], rcbuf.at[b1],
                                  rcsem.at[b1]).wait()
            pltpu.async_copy(x_hbm.at[rcbuf.at[b1, 1]], gbuf.at[b1],
                             gsem.at[b1])

        pltpu.make_async_copy(x_hbm.at[rcbuf.at[b0, 1]], gbuf.at[b0],
                              gsem.at[b0]).wait()
        pltpu.sync_copy(gbuf.at[b0], agg_sp.at[rcbuf.at[b0, 0]], add=True)
        return carry

    lax.fori_loop(0, NCHUNK, body, 0)

    plsc.subcore_barrier()

    pltpu.sync_copy(agg_sp.at[pl.ds(s * RSUB, RSUB)],
                    out_hbm.at[c, pl.ds(s * RSUB, RSUB)])

    @pl.when(s == NS - 1)
    def _():
        pltpu.sync_copy(agg_sp.at[pl.ds(NS * RSUB, N - NS * RSUB)],
                        out_hbm.at[c, pl.ds(NS * RSUB, N - NS * RSUB)])


def _agg(x_ext, x, rc4, zero):
    mesh = plsc.VectorSubcoreMesh(core_axis_name="c", subcore_axis_name="s")
    return pl.kernel(
        _agg_body,
        out_type=jax.ShapeDtypeStruct((NC, N, D), jnp.float32),
        mesh=mesh,
        scratch_types=[
            pltpu.VMEM((NBUF, 2, K), jnp.int32),
            pltpu.VMEM((NBUF, K, D), jnp.float32),
            pltpu.SemaphoreType.DMA((NBUF,)),
            pltpu.SemaphoreType.DMA((NBUF,)),
            pltpu.VMEM_SHARED((N, D), jnp.float32),
        ],
    )(x_ext, x, rc4, zero)


def _mlp_body(agg_ref, w1_ref, b1_ref, w2_ref, b2_ref, g_ref, be_ref, o_ref):
    h = agg_ref[0] + agg_ref[1]
    h = jnp.dot(h, w1_ref[...], preferred_element_type=jnp.float32)
    h = jnp.maximum(h + b1_ref[...], 0.0)
    h = jnp.dot(h, w2_ref[...], preferred_element_type=jnp.float32)
    h = h + b2_ref[...]
    mean = jnp.mean(h, axis=0, keepdims=True)
    cen = h - mean
    var = jnp.mean(cen * cen, axis=0, keepdims=True)
    o_ref[...] = cen * lax.rsqrt(var + EPS) * g_ref[...] + be_ref[...]


def _mlp(agg, w1, b1, w2, b2, g, be):
    return pl.pallas_call(
        _mlp_body,
        out_shape=jax.ShapeDtypeStruct((N, D), jnp.float32),
    )(agg, w1, b1.reshape(1, D), w2, b2.reshape(1, D),
      g.reshape(1, D), be.reshape(1, D))


def kernel(x, edge_index,
           W1_0, b1_0, W2_0, b2_0, g_0, be_0,
           W1_1, b1_1, W2_1, b2_1, g_1, be_1,
           W1_2, b1_2, W2_2, b2_2, g_2, be_2):
    row = edge_index[0].reshape(NW, EDGES_PER_W)
    col = edge_index[1].reshape(NW, EDGES_PER_W)
    rowp = jnp.pad(row, ((0, 0), (0, PAD)),
                   constant_values=0).reshape(NW, NCHUNK, 1, K)
    colp = jnp.pad(col, ((0, 0), (0, PAD)),
                   constant_values=N).reshape(NW, NCHUNK, 1, K)
    rc4 = jnp.concatenate([rowp, colp], axis=2)
    zero = jnp.zeros((N, D), jnp.float32)
    zrows = jnp.zeros((8, D), jnp.float32)
    params = [
        (W1_0, b1_0, W2_0, b2_0, g_0, be_0),
        (W1_1, b1_1, W2_1, b2_1, g_1, be_1),
        (W1_2, b1_2, W2_2, b2_2, g_2, be_2),
    ]
    for (w1, b1, w2, b2, g, be) in params:
        x_ext = jnp.concatenate([x, zrows], axis=0)
        agg = _agg(x_ext, x, rc4, zero)
        x = _mlp(agg, w1, b1, w2, b2, g, be)
    return x
